# baseline (speedup 1.0000x reference)
import functools

import jax
import jax.numpy as jnp
from jax import lax
from jax.experimental import pallas as pl
from jax.experimental.pallas import tpu as pltpu
from jax.experimental.pallas import tpu_sc as plsc

VOCAB = 32000
HIDDEN = 64
B = 16384
L = 50

NC = 2
NS = 16
NW = NC * NS

B_PER_W = B // NW
HALF = 256
NSTEP = 2 * L
HH = HIDDEN // 8
BH = B // 128
BH_PER_W = B_PER_W // 128


def _zero_acc(acc_v):
    zero = jnp.zeros((16,), jnp.float32)

    def zrow(h, carry):
        for c in range(B_PER_W // 16):
            acc_v[h, pl.ds(c * 16, 16)] = zero
        return carry

    lax.fori_loop(0, HIDDEN, zrow, 0)


def _body(ids_hbm, table_hbm, lhs_hbm, pool_hbm,
          idx_v0, idx_v1, rows_v0, rows_v1, trans_v0, trans_v1, acc_v,
          sem_idx, sem_g, sem_st):
    cid = lax.axis_index("c")
    sid = lax.axis_index("s")
    wid = sid * NC + cid
    row0 = wid * BH_PER_W

    iotas = [jnp.arange(bq * 16, bq * 16 + 16, dtype=jnp.int32)
             for bq in range(HALF // 16)]

    def fire_idx(t, idx_p):
        l = t // 2
        h2 = t % 2
        pltpu.async_copy(
            ids_hbm.at[l, pl.ds(row0 + h2 * 2, 2)], idx_p, sem_idx)

    def drain_idx(t, idx_p):
        l = t // 2
        h2 = t % 2
        pltpu.make_async_copy(
            ids_hbm.at[l, pl.ds(row0 + h2 * 2, 2)], idx_p, sem_idx).wait()

    def fire_gathers(idx_p, rows_p):
        for j in range(2):
            pltpu.async_copy(
                table_hbm.at[idx_p.at[j]],
                rows_p.at[pl.ds(j * 128, 128)], sem_g)

    def drain_gathers(idx_p, rows_p):
        for j in range(2):
            pltpu.make_async_copy(
                table_hbm.at[idx_p.at[j]],
                rows_p.at[pl.ds(j * 128, 128)], sem_g).wait()

    def fire_stores(t, trans_p):
        l = t // 2
        col = row0 + (t % 2) * 2
        for hh in range(HH):
            pltpu.async_copy(
                trans_p.at[hh], lhs_hbm.at[l, hh, pl.ds(col, 2)], sem_st)

    def drain_stores(t, trans_p):
        l = t // 2
        col = row0 + (t % 2) * 2
        for hh in range(HH):
            pltpu.make_async_copy(
                trans_p.at[hh], lhs_hbm.at[l, hh, pl.ds(col, 2)], sem_st
            ).wait()

    def one_step(t, half, idx_p, idx_q, rows_p, rows_q, trans_p, trans_q):
        @pl.when(t >= 1)
        def _():
            drain_stores(t - 1, trans_q)

        drain_gathers(idx_p, rows_p)

        @pl.when(t + 2 < NSTEP)
        def _():
            fire_idx(t + 2, idx_p)

        @pl.when(t + 1 < NSTEP)
        def _():
            drain_idx(t + 1, idx_q)
            fire_gathers(idx_q, rows_q)

        acc_base = half * HALF

        def hrow(h, carry):
            hh = h // 8
            hl = h % 8
            hsplat = jnp.full((16,), h, jnp.int32)
            for bq in range(HALF // 16):
                v = plsc.load_gather(rows_p, [iotas[bq], hsplat])
                trans_p[hh, bq // 8, hl, pl.ds((bq % 8) * 16, 16)] = v
                plsc.addupdate(acc_v.at[h, pl.ds(acc_base + bq * 16, 16)], v)
            return carry

        lax.fori_loop(0, HIDDEN, hrow, 0)

        fire_stores(t, trans_p)

    _zero_acc(acc_v)

    fire_idx(0, idx_v0)
    drain_idx(0, idx_v0)
    fire_gathers(idx_v0, rows_v0)
    fire_idx(1, idx_v1)

    def body2(gg, carry):
        one_step(2 * gg, 0, idx_v0, idx_v1, rows_v0, rows_v1,
                 trans_v0, trans_v1)
        one_step(2 * gg + 1, 1, idx_v1, idx_v0, rows_v1, rows_v0,
                 trans_v1, trans_v0)
        return carry

    lax.fori_loop(0, NSTEP // 2, body2, 0)

    drain_stores(NSTEP - 1, trans_v1)

    inv_l = jnp.float32(1.0 / L)
    hiotas = [jnp.arange(hq * 16, hq * 16 + 16, dtype=jnp.int32)
              for hq in range(HIDDEN // 16)]

    for half in range(2):
        rows_p = rows_v0 if half == 0 else rows_v1

        def brow(bl, carry, half=half, rows_p=rows_p):
            bsplat = jnp.full((16,), half * HALF + bl, jnp.int32)
            for hq in range(HIDDEN // 16):
                v = plsc.load_gather(acc_v, [hiotas[hq], bsplat])
                rows_p[bl, pl.ds(hq * 16, 16)] = v * inv_l
            return carry

        lax.fori_loop(0, HALF, brow, 0)
        pltpu.sync_copy(
            rows_p,
            pool_hbm.at[pl.ds(wid * B_PER_W + half * HALF, HALF)])


@jax.jit
def _encode(ids3d, table):
    mesh = plsc.VectorSubcoreMesh(core_axis_name="c", subcore_axis_name="s")
    kern = functools.partial(
        pl.kernel,
        out_type=[
            jax.ShapeDtypeStruct((L, HH, BH, 8, 128), jnp.float32),
            jax.ShapeDtypeStruct((B, HIDDEN), jnp.float32),
        ],
        mesh=mesh,
        scratch_types=[
            pltpu.VMEM((2, 128), jnp.int32),
            pltpu.VMEM((2, 128), jnp.int32),
            pltpu.VMEM((HALF, HIDDEN), jnp.float32),
            pltpu.VMEM((HALF, HIDDEN), jnp.float32),
            pltpu.VMEM((HH, 2, 8, 128), jnp.float32),
            pltpu.VMEM((HH, 2, 8, 128), jnp.float32),
            pltpu.VMEM((HIDDEN, B_PER_W), jnp.float32),
            pltpu.SemaphoreType.DMA,
            pltpu.SemaphoreType.DMA,
            pltpu.SemaphoreType.DMA,
        ],
        compiler_params=pltpu.CompilerParams(
            use_tc_tiling_on_sc=False, needs_layout_passes=False),
    )(_body)
    return kern(ids3d, table)


def kernel(input_ids, embed_weight):
    ids3d = input_ids.astype(jnp.int32).T.reshape(L, BH, 128)
    lhs5d, pool = _encode(ids3d, embed_weight)
    lhs = lhs5d.transpose(2, 4, 0, 1, 3).reshape(B, L, HIDDEN)
    return (lhs, pool)


# speedup vs baseline: 1.2160x; 1.2160x over previous
import functools

import jax
import jax.numpy as jnp
from jax import lax
from jax.experimental import pallas as pl
from jax.experimental.pallas import tpu as pltpu
from jax.experimental.pallas import tpu_sc as plsc

VOCAB = 32000
HIDDEN = 64
B = 16384
L = 50

NC = 2
NS = 16
NW = NC * NS

B_PER_W = B // NW
HALF = 256
NSTEP = 2 * L
HH = HIDDEN // 8
BH = B // 128
BH_PER_W = B_PER_W // 128


def _zero_acc(acc_v):
    zero = jnp.zeros((16,), jnp.float32)

    def zrow(h, carry):
        for c in range(B_PER_W // 16):
            acc_v[h, pl.ds(c * 16, 16)] = zero
        return carry

    lax.fori_loop(0, HIDDEN, zrow, 0)


def _body(ids_hbm, table_hbm, lhs_hbm, pool_hbm,
          idx_v0, idx_v1, rows_v0, rows_v1, trans_v0, trans_v1, acc_v,
          sem_idx, sem_g, sem_st):
    cid = lax.axis_index("c")
    sid = lax.axis_index("s")
    wid = sid * NC + cid
    row0 = wid * BH_PER_W

    iotas = [jnp.arange(bq * 16, bq * 16 + 16, dtype=jnp.int32)
             for bq in range(HALF // 16)]

    def fire_idx(t, idx_p):
        l = t // 2
        h2 = t % 2
        pltpu.async_copy(
            ids_hbm.at[l, pl.ds(row0 + h2 * 2, 2)], idx_p, sem_idx)

    def drain_idx(t, idx_p):
        l = t // 2
        h2 = t % 2
        pltpu.make_async_copy(
            ids_hbm.at[l, pl.ds(row0 + h2 * 2, 2)], idx_p, sem_idx).wait()

    def fire_gathers(idx_p, rows_p):
        for j in range(2):
            pltpu.async_copy(
                table_hbm.at[idx_p.at[j]],
                rows_p.at[pl.ds(j * 128, 128)], sem_g)

    def drain_gathers(idx_p, rows_p):
        for j in range(2):
            pltpu.make_async_copy(
                table_hbm.at[idx_p.at[j]],
                rows_p.at[pl.ds(j * 128, 128)], sem_g).wait()

    def fire_stores(t, trans_p):
        l = t // 2
        col = row0 + (t % 2) * 2
        for hh in range(HH):
            pltpu.async_copy(
                trans_p.at[hh], lhs_hbm.at[l, hh, pl.ds(col, 2)], sem_st)

    def drain_stores(t, trans_p):
        l = t // 2
        col = row0 + (t % 2) * 2
        for hh in range(HH):
            pltpu.make_async_copy(
                trans_p.at[hh], lhs_hbm.at[l, hh, pl.ds(col, 2)], sem_st
            ).wait()

    def one_step(t, half, idx_p, idx_q, rows_p, rows_q, trans_p, trans_q):
        @pl.when(t >= 1)
        def _():
            drain_stores(t - 1, trans_q)

        drain_gathers(idx_p, rows_p)

        @pl.when(t + 2 < NSTEP)
        def _():
            fire_idx(t + 2, idx_p)

        @pl.when(t + 1 < NSTEP)
        def _():
            drain_idx(t + 1, idx_q)
            fire_gathers(idx_q, rows_q)

        acc_base = half * HALF

        def hrow(h, carry):
            hh = h // 8
            hl = h % 8
            hsplat = jnp.full((16,), h, jnp.int32)
            vs = [plsc.load_gather(rows_p, [iotas[bq], hsplat])
                  for bq in range(HALF // 16)]
            for bq, v in enumerate(vs):
                trans_p[hh, bq // 8, hl, pl.ds((bq % 8) * 16, 16)] = v
            for bq, v in enumerate(vs):
                plsc.addupdate(acc_v.at[h, pl.ds(acc_base + bq * 16, 16)], v)
            return carry

        lax.fori_loop(0, HIDDEN, hrow, 0)

        fire_stores(t, trans_p)

    _zero_acc(acc_v)

    fire_idx(0, idx_v0)
    drain_idx(0, idx_v0)
    fire_gathers(idx_v0, rows_v0)
    fire_idx(1, idx_v1)

    def body2(gg, carry):
        one_step(2 * gg, 0, idx_v0, idx_v1, rows_v0, rows_v1,
                 trans_v0, trans_v1)
        one_step(2 * gg + 1, 1, idx_v1, idx_v0, rows_v1, rows_v0,
                 trans_v1, trans_v0)
        return carry

    lax.fori_loop(0, NSTEP // 2, body2, 0)

    drain_stores(NSTEP - 1, trans_v1)

    inv_l = jnp.float32(1.0 / L)
    hiotas = [jnp.arange(hq * 16, hq * 16 + 16, dtype=jnp.int32)
              for hq in range(HIDDEN // 16)]

    for half in range(2):
        rows_p = rows_v0 if half == 0 else rows_v1

        def brow(bl, carry, half=half, rows_p=rows_p):
            bsplat = jnp.full((16,), half * HALF + bl, jnp.int32)
            for hq in range(HIDDEN // 16):
                v = plsc.load_gather(acc_v, [hiotas[hq], bsplat])
                rows_p[bl, pl.ds(hq * 16, 16)] = v * inv_l
            return carry

        lax.fori_loop(0, HALF, brow, 0)
        pltpu.sync_copy(
            rows_p,
            pool_hbm.at[pl.ds(wid * B_PER_W + half * HALF, HALF)])


@jax.jit
def _encode(ids3d, table):
    mesh = plsc.VectorSubcoreMesh(core_axis_name="c", subcore_axis_name="s")
    kern = functools.partial(
        pl.kernel,
        out_type=[
            jax.ShapeDtypeStruct((L, HH, BH, 8, 128), jnp.float32),
            jax.ShapeDtypeStruct((B, HIDDEN), jnp.float32),
        ],
        mesh=mesh,
        scratch_types=[
            pltpu.VMEM((2, 128), jnp.int32),
            pltpu.VMEM((2, 128), jnp.int32),
            pltpu.VMEM((HALF, HIDDEN), jnp.float32),
            pltpu.VMEM((HALF, HIDDEN), jnp.float32),
            pltpu.VMEM((HH, 2, 8, 128), jnp.float32),
            pltpu.VMEM((HH, 2, 8, 128), jnp.float32),
            pltpu.VMEM((HIDDEN, B_PER_W), jnp.float32),
            pltpu.SemaphoreType.DMA,
            pltpu.SemaphoreType.DMA,
            pltpu.SemaphoreType.DMA,
        ],
        compiler_params=pltpu.CompilerParams(
            use_tc_tiling_on_sc=False, needs_layout_passes=False),
    )(_body)
    return kern(ids3d, table)


def kernel(input_ids, embed_weight):
    ids3d = input_ids.astype(jnp.int32).T.reshape(L, BH, 128)
    lhs5d, pool = _encode(ids3d, embed_weight)
    lhs = lhs5d.transpose(2, 4, 0, 1, 3).reshape(B, L, HIDDEN)
    return (lhs, pool)


# speedup vs baseline: 1.7169x; 1.4120x over previous
import functools

import jax
import jax.numpy as jnp
from jax import lax
from jax.experimental import pallas as pl
from jax.experimental.pallas import tpu as pltpu
from jax.experimental.pallas import tpu_sc as plsc

VOCAB = 32000
HIDDEN = 64
B = 16384
L = 50

NC = 2
NS = 16
NW = NC * NS

B_PER_W = B // NW
HALF = 256
NSTEP = 2 * L
HH = HIDDEN // 8
BH = B // 128
BH_PER_W = B_PER_W // 128


def _zero_acc(acc_v):
    zero = jnp.zeros((16,), jnp.float32)

    def zrow(h, carry):
        for c in range(B_PER_W // 16):
            acc_v[h, pl.ds(c * 16, 16)] = zero
        return carry

    lax.fori_loop(0, HIDDEN, zrow, 0)


def _body(ids_hbm, table_hbm, lhs_hbm, pool_hbm,
          idx_v0, idx_v1, rows_v0, rows_v1, trans_v0, trans_v1, staged_v,
          acc_v, sem_idx, sem_g, sem_st):
    cid = lax.axis_index("c")
    sid = lax.axis_index("s")
    wid = sid * NC + cid
    row0 = wid * BH_PER_W

    iotas = [jnp.arange(bq * 16, bq * 16 + 16, dtype=jnp.int32)
             for bq in range(HALF // 16)]

    def fire_idx(t, idx_p):
        l = t // 2
        h2 = t % 2
        pltpu.async_copy(
            ids_hbm.at[l, pl.ds(row0 + h2 * 2, 2)], idx_p, sem_idx)

    def drain_idx(t, idx_p):
        l = t // 2
        h2 = t % 2
        pltpu.make_async_copy(
            ids_hbm.at[l, pl.ds(row0 + h2 * 2, 2)], idx_p, sem_idx).wait()

    def fire_gathers(idx_p, rows_p):
        for j in range(2):
            pltpu.async_copy(
                table_hbm.at[idx_p.at[j]],
                rows_p.at[pl.ds(j * 128, 128)], sem_g)

    def drain_gathers(idx_p, rows_p):
        for j in range(2):
            pltpu.make_async_copy(
                table_hbm.at[idx_p.at[j]],
                rows_p.at[pl.ds(j * 128, 128)], sem_g).wait()

    def fire_stores(t, trans_p):
        l = t // 2
        col = row0 + (t % 2) * 2
        for hh in range(HH):
            pltpu.async_copy(
                trans_p.at[hh], lhs_hbm.at[l, hh, pl.ds(col, 2)], sem_st)

    def drain_stores(t, trans_p):
        l = t // 2
        col = row0 + (t % 2) * 2
        for hh in range(HH):
            pltpu.make_async_copy(
                trans_p.at[hh], lhs_hbm.at[l, hh, pl.ds(col, 2)], sem_st
            ).wait()

    def one_step(t, half, idx_p, idx_q, rows_p, rows_q, trans_p, trans_q):
        @pl.when(t >= 1)
        def _():
            drain_stores(t - 1, trans_q)

        drain_gathers(idx_p, rows_p)

        @pl.when(t + 2 < NSTEP)
        def _():
            fire_idx(t + 2, idx_p)

        @pl.when(t + 1 < NSTEP)
        def _():
            drain_idx(t + 1, idx_q)
            fire_gathers(idx_q, rows_q)

        def srow(r4, carry):
            for u in range(4):
                r = r4 * 4 + u
                for c in range(HIDDEN // 16):
                    staged_v[r, pl.ds(c * 16, 16)] = rows_p[r, pl.ds(c * 16, 16)]
            return carry

        lax.fori_loop(0, HALF // 4, srow, 0)

        acc_base = half * HALF

        def hrow(h, carry):
            hh = h // 8
            hl = h % 8
            hsplat = jnp.full((16,), h, jnp.int32)
            vs = [plsc.load_gather(staged_v, [iotas[bq], hsplat])
                  for bq in range(HALF // 16)]
            for bq, v in enumerate(vs):
                trans_p[hh, bq // 8, hl, pl.ds((bq % 8) * 16, 16)] = v
            for bq, v in enumerate(vs):
                plsc.addupdate(acc_v.at[h, pl.ds(acc_base + bq * 16, 16)], v)
            return carry

        lax.fori_loop(0, HIDDEN, hrow, 0)

        fire_stores(t, trans_p)

    _zero_acc(acc_v)

    fire_idx(0, idx_v0)
    drain_idx(0, idx_v0)
    fire_gathers(idx_v0, rows_v0)
    fire_idx(1, idx_v1)

    def body2(gg, carry):
        one_step(2 * gg, 0, idx_v0, idx_v1, rows_v0, rows_v1,
                 trans_v0, trans_v1)
        one_step(2 * gg + 1, 1, idx_v1, idx_v0, rows_v1, rows_v0,
                 trans_v1, trans_v0)
        return carry

    lax.fori_loop(0, NSTEP // 2, body2, 0)

    drain_stores(NSTEP - 1, trans_v1)

    inv_l = jnp.float32(1.0 / L)
    hiotas = [jnp.arange(hq * 16, hq * 16 + 16, dtype=jnp.int32)
              for hq in range(HIDDEN // 16)]

    for half in range(2):
        rows_p = rows_v0 if half == 0 else rows_v1

        def brow(bl, carry, half=half, rows_p=rows_p):
            bsplat = jnp.full((16,), half * HALF + bl, jnp.int32)
            for hq in range(HIDDEN // 16):
                v = plsc.load_gather(acc_v, [hiotas[hq], bsplat])
                rows_p[bl, pl.ds(hq * 16, 16)] = v * inv_l
            return carry

        lax.fori_loop(0, HALF, brow, 0)
        pltpu.sync_copy(
            rows_p,
            pool_hbm.at[pl.ds(wid * B_PER_W + half * HALF, HALF)])


@jax.jit
def _encode(ids3d, table):
    mesh = plsc.VectorSubcoreMesh(core_axis_name="c", subcore_axis_name="s")
    kern = functools.partial(
        pl.kernel,
        out_type=[
            jax.ShapeDtypeStruct((L, HH, BH, 8, 128), jnp.float32),
            jax.ShapeDtypeStruct((B, HIDDEN), jnp.float32),
        ],
        mesh=mesh,
        scratch_types=[
            pltpu.VMEM((2, 128), jnp.int32),
            pltpu.VMEM((2, 128), jnp.int32),
            pltpu.VMEM((HALF, HIDDEN), jnp.float32),
            pltpu.VMEM((HALF, HIDDEN), jnp.float32),
            pltpu.VMEM((HH, 2, 8, 128), jnp.float32),
            pltpu.VMEM((HH, 2, 8, 128), jnp.float32),
            pltpu.VMEM((HALF, HIDDEN + 1), jnp.float32),
            pltpu.VMEM((HIDDEN, B_PER_W), jnp.float32),
            pltpu.SemaphoreType.DMA,
            pltpu.SemaphoreType.DMA,
            pltpu.SemaphoreType.DMA,
        ],
        compiler_params=pltpu.CompilerParams(
            use_tc_tiling_on_sc=False, needs_layout_passes=False),
    )(_body)
    return kern(ids3d, table)


def kernel(input_ids, embed_weight):
    ids3d = input_ids.astype(jnp.int32).T.reshape(L, BH, 128)
    lhs5d, pool = _encode(ids3d, embed_weight)
    lhs = lhs5d.transpose(2, 4, 0, 1, 3).reshape(B, L, HIDDEN)
    return (lhs, pool)


# speedup vs baseline: 3.9598x; 2.3063x over previous
import functools

import jax
import jax.numpy as jnp
from jax import lax
from jax.experimental import pallas as pl
from jax.experimental.pallas import tpu as pltpu
from jax.experimental.pallas import tpu_sc as plsc

VOCAB = 32000
HIDDEN = 64
B = 16384
L = 50

NC = 2
NS = 16
NW = NC * NS

B_PER_W = B // NW
HALF = 256
NSTEP = 2 * L
HH = HIDDEN // 8
BH = B // 128
BH_PER_W = B_PER_W // 128


def _zero_acc(acc_v):
    zero = jnp.zeros((16,), jnp.float32)

    def zrow(h, carry):
        for c in range(B_PER_W // 16):
            acc_v[h, pl.ds(c * 16, 16)] = zero
        return carry

    lax.fori_loop(0, HIDDEN, zrow, 0)


def _body(ids_hbm, table_hbm, lhs_hbm, pool_hbm,
          idx_v0, idx_v1, rows_v0, rows_v1, trans_v0, trans_v1, staged_v,
          acc_v, sem_idx, sem_g, sem_st):
    cid = lax.axis_index("c")
    sid = lax.axis_index("s")
    wid = sid * NC + cid
    row0 = wid * BH_PER_W

    PITCH = HIDDEN + 1
    iota65 = [jnp.arange(bq * 16, bq * 16 + 16, dtype=jnp.int32) * PITCH
              for bq in range(HALF // 16)]

    def fire_idx(t, idx_p):
        l = t // 2
        h2 = t % 2
        pltpu.async_copy(
            ids_hbm.at[l, pl.ds(row0 + h2 * 2, 2)], idx_p, sem_idx)

    def drain_idx(t, idx_p):
        l = t // 2
        h2 = t % 2
        pltpu.make_async_copy(
            ids_hbm.at[l, pl.ds(row0 + h2 * 2, 2)], idx_p, sem_idx).wait()

    def fire_gathers(idx_p, rows_p):
        for j in range(2):
            pltpu.async_copy(
                table_hbm.at[idx_p.at[j]],
                rows_p.at[pl.ds(j * 128, 128)], sem_g)

    def drain_gathers(idx_p, rows_p):
        for j in range(2):
            pltpu.make_async_copy(
                table_hbm.at[idx_p.at[j]],
                rows_p.at[pl.ds(j * 128, 128)], sem_g).wait()

    def fire_stores(t, trans_p):
        l = t // 2
        col = row0 + (t % 2) * 2
        for hh in range(HH):
            pltpu.async_copy(
                trans_p.at[hh], lhs_hbm.at[l, hh, pl.ds(col, 2)], sem_st)

    def drain_stores(t, trans_p):
        l = t // 2
        col = row0 + (t % 2) * 2
        for hh in range(HH):
            pltpu.make_async_copy(
                trans_p.at[hh], lhs_hbm.at[l, hh, pl.ds(col, 2)], sem_st
            ).wait()

    def one_step(t, half, idx_p, idx_q, rows_p, rows_q, trans_p, trans_q):
        @pl.when(t >= 1)
        def _():
            drain_stores(t - 1, trans_q)

        drain_gathers(idx_p, rows_p)

        @pl.when(t + 2 < NSTEP)
        def _():
            fire_idx(t + 2, idx_p)

        @pl.when(t + 1 < NSTEP)
        def _():
            drain_idx(t + 1, idx_q)
            fire_gathers(idx_q, rows_q)

        @plsc.parallel_loop(0, HALF // 4, unroll=2)
        def srow(r4):
            for u in range(4):
                r = r4 * 4 + u
                base = r * PITCH
                for c in range(HIDDEN // 16):
                    staged_v[pl.ds(base + c * 16, 16)] = (
                        rows_p[r, pl.ds(c * 16, 16)])

        acc_base = half * HALF

        @plsc.parallel_loop(0, HIDDEN, unroll=2)
        def hrow(h):
            hh = h // 8
            hl = h % 8
            hsplat = jnp.full((16,), h, jnp.int32)
            vs = [plsc.load_gather(staged_v, [iota65[bq] + hsplat])
                  for bq in range(HALF // 16)]
            for bq, v in enumerate(vs):
                trans_p[hh, bq // 8, hl, pl.ds((bq % 8) * 16, 16)] = v
            for bq, v in enumerate(vs):
                plsc.addupdate(acc_v.at[h, pl.ds(acc_base + bq * 16, 16)], v)

        fire_stores(t, trans_p)

    _zero_acc(acc_v)

    fire_idx(0, idx_v0)
    drain_idx(0, idx_v0)
    fire_gathers(idx_v0, rows_v0)
    fire_idx(1, idx_v1)

    def body2(gg, carry):
        one_step(2 * gg, 0, idx_v0, idx_v1, rows_v0, rows_v1,
                 trans_v0, trans_v1)
        one_step(2 * gg + 1, 1, idx_v1, idx_v0, rows_v1, rows_v0,
                 trans_v1, trans_v0)
        return carry

    lax.fori_loop(0, NSTEP // 2, body2, 0)

    drain_stores(NSTEP - 1, trans_v1)

    inv_l = jnp.float32(1.0 / L)
    hiotas = [jnp.arange(hq * 16, hq * 16 + 16, dtype=jnp.int32)
              for hq in range(HIDDEN // 16)]

    for half in range(2):
        rows_p = rows_v0 if half == 0 else rows_v1

        def brow(bl, carry, half=half, rows_p=rows_p):
            bsplat = jnp.full((16,), half * HALF + bl, jnp.int32)
            for hq in range(HIDDEN // 16):
                v = plsc.load_gather(acc_v, [hiotas[hq], bsplat])
                rows_p[bl, pl.ds(hq * 16, 16)] = v * inv_l
            return carry

        lax.fori_loop(0, HALF, brow, 0)
        pltpu.sync_copy(
            rows_p,
            pool_hbm.at[pl.ds(wid * B_PER_W + half * HALF, HALF)])


@jax.jit
def _encode(ids3d, table):
    mesh = plsc.VectorSubcoreMesh(core_axis_name="c", subcore_axis_name="s")
    kern = functools.partial(
        pl.kernel,
        out_type=[
            jax.ShapeDtypeStruct((L, HH, BH, 8, 128), jnp.float32),
            jax.ShapeDtypeStruct((B, HIDDEN), jnp.float32),
        ],
        mesh=mesh,
        scratch_types=[
            pltpu.VMEM((2, 128), jnp.int32),
            pltpu.VMEM((2, 128), jnp.int32),
            pltpu.VMEM((HALF, HIDDEN), jnp.float32),
            pltpu.VMEM((HALF, HIDDEN), jnp.float32),
            pltpu.VMEM((HH, 2, 8, 128), jnp.float32),
            pltpu.VMEM((HH, 2, 8, 128), jnp.float32),
            pltpu.VMEM((HALF * (HIDDEN + 1),), jnp.float32),
            pltpu.VMEM((HIDDEN, B_PER_W), jnp.float32),
            pltpu.SemaphoreType.DMA,
            pltpu.SemaphoreType.DMA,
            pltpu.SemaphoreType.DMA,
        ],
        compiler_params=pltpu.CompilerParams(
            use_tc_tiling_on_sc=False, needs_layout_passes=False),
    )(_body)
    return kern(ids3d, table)


def kernel(input_ids, embed_weight):
    ids3d = input_ids.astype(jnp.int32).T.reshape(L, BH, 128)
    lhs5d, pool = _encode(ids3d, embed_weight)
    lhs = lhs5d.transpose(2, 4, 0, 1, 3).reshape(B, L, HIDDEN)
    return (lhs, pool)


# speedup vs baseline: 4.2473x; 1.0726x over previous
import functools

import jax
import jax.numpy as jnp
from jax import lax
from jax.experimental import pallas as pl
from jax.experimental.pallas import tpu as pltpu
from jax.experimental.pallas import tpu_sc as plsc

VOCAB = 32000
HIDDEN = 64
B = 16384
L = 50

NC = 2
NS = 16
NW = NC * NS

B_PER_W = B // NW
HALF = 256
NSTEP = 2 * L
HH = HIDDEN // 8
BH = B // 128
BH_PER_W = B_PER_W // 128


def _zero_acc(acc_v):
    zero = jnp.zeros((16,), jnp.float32)

    def zrow(h, carry):
        for c in range(B_PER_W // 16):
            acc_v[h, pl.ds(c * 16, 16)] = zero
        return carry

    lax.fori_loop(0, HIDDEN, zrow, 0)


def _body(ids_hbm, table_hbm, lhs_hbm, pool_hbm,
          idx_v0, idx_v1, rows_v0, rows_v1, trans_v0, trans_v1, staged_v,
          acc_v, sem_idx, sem_g, sem_st):
    cid = lax.axis_index("c")
    sid = lax.axis_index("s")
    wid = sid * NC + cid
    row0 = wid * BH_PER_W

    PITCH = HIDDEN + 1
    iota65 = [jnp.arange(bq * 16, bq * 16 + 16, dtype=jnp.int32) * PITCH
              for bq in range(HALF // 16)]

    def fire_idx(t, idx_p):
        l = t // 2
        h2 = t % 2
        pltpu.async_copy(
            ids_hbm.at[l, pl.ds(row0 + h2 * 2, 2)], idx_p, sem_idx)

    def drain_idx(t, idx_p):
        l = t // 2
        h2 = t % 2
        pltpu.make_async_copy(
            ids_hbm.at[l, pl.ds(row0 + h2 * 2, 2)], idx_p, sem_idx).wait()

    def fire_gathers(idx_p, rows_p):
        for j in range(2):
            pltpu.async_copy(
                table_hbm.at[idx_p.at[j]],
                rows_p.at[pl.ds(j * 128, 128)], sem_g)

    def drain_gathers(idx_p, rows_p):
        for j in range(2):
            pltpu.make_async_copy(
                table_hbm.at[idx_p.at[j]],
                rows_p.at[pl.ds(j * 128, 128)], sem_g).wait()

    def fire_stores(t, trans_p):
        l = t // 2
        col = row0 + (t % 2) * 2
        for hh in range(HH):
            pltpu.async_copy(
                trans_p.at[hh], lhs_hbm.at[l, hh, pl.ds(col, 2)], sem_st)

    def drain_stores(t, trans_p):
        l = t // 2
        col = row0 + (t % 2) * 2
        for hh in range(HH):
            pltpu.make_async_copy(
                trans_p.at[hh], lhs_hbm.at[l, hh, pl.ds(col, 2)], sem_st
            ).wait()

    def one_step(t, half, idx_p, idx_q, rows_p, rows_q, trans_p, trans_q):
        @pl.when(t >= 1)
        def _():
            drain_stores(t - 1, trans_q)

        drain_gathers(idx_p, rows_p)

        @pl.when(t + 2 < NSTEP)
        def _():
            fire_idx(t + 2, idx_p)

        @pl.when(t + 1 < NSTEP)
        def _():
            drain_idx(t + 1, idx_q)
            fire_gathers(idx_q, rows_q)

        @plsc.parallel_loop(0, HALF // 4, unroll=4)
        def srow(r4):
            for u in range(4):
                r = r4 * 4 + u
                base = r * PITCH
                for c in range(HIDDEN // 16):
                    staged_v[pl.ds(base + c * 16, 16)] = (
                        rows_p[r, pl.ds(c * 16, 16)])

        acc_base = half * HALF

        @plsc.parallel_loop(0, HIDDEN, unroll=4)
        def hrow(h):
            hh = h // 8
            hl = h % 8
            hsplat = jnp.full((16,), h, jnp.int32)
            vs = [plsc.load_gather(staged_v, [iota65[bq] + hsplat])
                  for bq in range(HALF // 16)]
            for bq, v in enumerate(vs):
                trans_p[hh, bq // 8, hl, pl.ds((bq % 8) * 16, 16)] = v
            for bq, v in enumerate(vs):
                plsc.addupdate(acc_v.at[h, pl.ds(acc_base + bq * 16, 16)], v)

        fire_stores(t, trans_p)

    _zero_acc(acc_v)

    fire_idx(0, idx_v0)
    drain_idx(0, idx_v0)
    fire_gathers(idx_v0, rows_v0)
    fire_idx(1, idx_v1)

    def body2(gg, carry):
        one_step(2 * gg, 0, idx_v0, idx_v1, rows_v0, rows_v1,
                 trans_v0, trans_v1)
        one_step(2 * gg + 1, 1, idx_v1, idx_v0, rows_v1, rows_v0,
                 trans_v1, trans_v0)
        return carry

    lax.fori_loop(0, NSTEP // 2, body2, 0)

    drain_stores(NSTEP - 1, trans_v1)

    inv_l = jnp.float32(1.0 / L)
    hiotas = [jnp.arange(hq * 16, hq * 16 + 16, dtype=jnp.int32)
              for hq in range(HIDDEN // 16)]

    for half in range(2):
        rows_p = rows_v0 if half == 0 else rows_v1

        @plsc.parallel_loop(0, HALF, unroll=4)
        def brow(bl, half=half, rows_p=rows_p):
            bsplat = jnp.full((16,), half * HALF + bl, jnp.int32)
            vs = [plsc.load_gather(acc_v, [hiotas[hq], bsplat])
                  for hq in range(HIDDEN // 16)]
            for hq, v in enumerate(vs):
                rows_p[bl, pl.ds(hq * 16, 16)] = v * inv_l
        pltpu.sync_copy(
            rows_p,
            pool_hbm.at[pl.ds(wid * B_PER_W + half * HALF, HALF)])


@jax.jit
def _encode(ids3d, table):
    mesh = plsc.VectorSubcoreMesh(core_axis_name="c", subcore_axis_name="s")
    kern = functools.partial(
        pl.kernel,
        out_type=[
            jax.ShapeDtypeStruct((L, HH, BH, 8, 128), jnp.float32),
            jax.ShapeDtypeStruct((B, HIDDEN), jnp.float32),
        ],
        mesh=mesh,
        scratch_types=[
            pltpu.VMEM((2, 128), jnp.int32),
            pltpu.VMEM((2, 128), jnp.int32),
            pltpu.VMEM((HALF, HIDDEN), jnp.float32),
            pltpu.VMEM((HALF, HIDDEN), jnp.float32),
            pltpu.VMEM((HH, 2, 8, 128), jnp.float32),
            pltpu.VMEM((HH, 2, 8, 128), jnp.float32),
            pltpu.VMEM((HALF * (HIDDEN + 1),), jnp.float32),
            pltpu.VMEM((HIDDEN, B_PER_W), jnp.float32),
            pltpu.SemaphoreType.DMA,
            pltpu.SemaphoreType.DMA,
            pltpu.SemaphoreType.DMA,
        ],
        compiler_params=pltpu.CompilerParams(
            use_tc_tiling_on_sc=False, needs_layout_passes=False),
    )(_body)
    return kern(ids3d, table)


def kernel(input_ids, embed_weight):
    ids3d = input_ids.astype(jnp.int32).T.reshape(L, BH, 128)
    lhs5d, pool = _encode(ids3d, embed_weight)
    lhs = lhs5d.transpose(2, 4, 0, 1, 3).reshape(B, L, HIDDEN)
    return (lhs, pool)


# speedup vs baseline: 4.6029x; 1.0837x over previous
import functools

import jax
import jax.numpy as jnp
from jax import lax
from jax.experimental import pallas as pl
from jax.experimental.pallas import tpu as pltpu
from jax.experimental.pallas import tpu_sc as plsc

VOCAB = 32000
HIDDEN = 64
B = 16384
L = 50

NC = 2
NS = 16
NW = NC * NS

B_PER_W = B // NW
HALF = 256
NSTEP = 2 * L
HH = HIDDEN // 8
BH = B // 128
BH_PER_W = B_PER_W // 128


def _zero_acc(acc_v):
    zero = jnp.zeros((16,), jnp.float32)

    def zrow(h, carry):
        hh = h // 8
        hl = h % 8
        for bh in range(BH_PER_W):
            for c in range(8):
                acc_v[hh, bh, hl, pl.ds(c * 16, 16)] = zero
        return carry

    lax.fori_loop(0, HIDDEN, zrow, 0)


def _body(ids_hbm, table_hbm, lhs_hbm, pool_hbm,
          idx_v0, idx_v1, rows_v0, rows_v1, trans_v0, trans_v1, staged_v,
          acc_v, sem_idx, sem_g, sem_st):
    cid = lax.axis_index("c")
    sid = lax.axis_index("s")
    wid = sid * NC + cid
    row0 = wid * BH_PER_W

    PITCH = HIDDEN + 1
    iota65 = [jnp.arange(bq * 16, bq * 16 + 16, dtype=jnp.int32) * PITCH
              for bq in range(HALF // 16)]

    def fire_idx(t, idx_p):
        l = t // 2
        h2 = t % 2
        pltpu.async_copy(
            ids_hbm.at[l, pl.ds(row0 + h2 * 2, 2)], idx_p, sem_idx)

    def drain_idx(t, idx_p):
        l = t // 2
        h2 = t % 2
        pltpu.make_async_copy(
            ids_hbm.at[l, pl.ds(row0 + h2 * 2, 2)], idx_p, sem_idx).wait()

    def fire_gathers(idx_p, rows_p):
        for j in range(2):
            pltpu.async_copy(
                table_hbm.at[idx_p.at[j]],
                rows_p.at[pl.ds(j * 128, 128)], sem_g)

    def drain_gathers(idx_p, rows_p):
        for j in range(2):
            pltpu.make_async_copy(
                table_hbm.at[idx_p.at[j]],
                rows_p.at[pl.ds(j * 128, 128)], sem_g).wait()

    def fire_stores(t, trans_p):
        l = t // 2
        col = row0 + (t % 2) * 2
        for hh in range(HH):
            pltpu.async_copy(
                trans_p.at[hh], lhs_hbm.at[l, hh, pl.ds(col, 2)], sem_st)

    def drain_stores(t, trans_p):
        l = t // 2
        col = row0 + (t % 2) * 2
        for hh in range(HH):
            pltpu.make_async_copy(
                trans_p.at[hh], lhs_hbm.at[l, hh, pl.ds(col, 2)], sem_st
            ).wait()

    def one_step(t, half, idx_p, idx_q, rows_p, rows_q, trans_p, trans_q):
        @pl.when(t >= 1)
        def _():
            drain_stores(t - 1, trans_q)

        drain_gathers(idx_p, rows_p)

        @pl.when(t + 2 < NSTEP)
        def _():
            fire_idx(t + 2, idx_p)

        @pl.when(t + 1 < NSTEP)
        def _():
            drain_idx(t + 1, idx_q)
            fire_gathers(idx_q, rows_q)

        @plsc.parallel_loop(0, HALF // 4, unroll=4)
        def srow(r4):
            for u in range(4):
                r = r4 * 4 + u
                base = r * PITCH
                for c in range(HIDDEN // 16):
                    staged_v[pl.ds(base + c * 16, 16)] = (
                        rows_p[r, pl.ds(c * 16, 16)])

        acc_bh = half * 2

        @plsc.parallel_loop(0, HIDDEN, unroll=4)
        def hrow(h):
            hh = h // 8
            hl = h % 8
            hsplat = jnp.full((16,), h, jnp.int32)
            vs = [plsc.load_gather(staged_v, [iota65[bq] + hsplat])
                  for bq in range(HALF // 16)]
            for bq, v in enumerate(vs):
                trans_p[hh, bq // 8, hl, pl.ds((bq % 8) * 16, 16)] = v
            for bq, v in enumerate(vs):
                plsc.addupdate(
                    acc_v.at[hh, acc_bh + bq // 8, hl,
                             pl.ds((bq % 8) * 16, 16)], v)

        fire_stores(t, trans_p)

    _zero_acc(acc_v)

    fire_idx(0, idx_v0)
    drain_idx(0, idx_v0)
    fire_gathers(idx_v0, rows_v0)
    fire_idx(1, idx_v1)

    def body2(gg, carry):
        one_step(2 * gg, 0, idx_v0, idx_v1, rows_v0, rows_v1,
                 trans_v0, trans_v1)
        one_step(2 * gg + 1, 1, idx_v1, idx_v0, rows_v1, rows_v0,
                 trans_v1, trans_v0)
        return carry

    lax.fori_loop(0, NSTEP // 2, body2, 0)

    drain_stores(NSTEP - 1, trans_v1)

    inv_l = jnp.float32(1.0 / L)

    @plsc.parallel_loop(0, HIDDEN, unroll=4)
    def prow(h):
        hh = h // 8
        hl = h % 8
        for bh in range(BH_PER_W):
            for c in range(8):
                sl = acc_v.at[hh, bh, hl, pl.ds(c * 16, 16)]
                sl[...] = sl[...] * inv_l

    for hh in range(HH):
        pltpu.sync_copy(
            acc_v.at[hh], pool_hbm.at[hh, pl.ds(wid * BH_PER_W, BH_PER_W)])


@jax.jit
def _encode(ids3d, table):
    mesh = plsc.VectorSubcoreMesh(core_axis_name="c", subcore_axis_name="s")
    kern = functools.partial(
        pl.kernel,
        out_type=[
            jax.ShapeDtypeStruct((L, HH, BH, 8, 128), jnp.float32),
            jax.ShapeDtypeStruct((HH, BH, 8, 128), jnp.float32),
        ],
        mesh=mesh,
        scratch_types=[
            pltpu.VMEM((2, 128), jnp.int32),
            pltpu.VMEM((2, 128), jnp.int32),
            pltpu.VMEM((HALF, HIDDEN), jnp.float32),
            pltpu.VMEM((HALF, HIDDEN), jnp.float32),
            pltpu.VMEM((HH, 2, 8, 128), jnp.float32),
            pltpu.VMEM((HH, 2, 8, 128), jnp.float32),
            pltpu.VMEM((HALF * (HIDDEN + 1),), jnp.float32),
            pltpu.VMEM((HH, BH_PER_W, 8, 128), jnp.float32),
            pltpu.SemaphoreType.DMA,
            pltpu.SemaphoreType.DMA,
            pltpu.SemaphoreType.DMA,
        ],
        compiler_params=pltpu.CompilerParams(
            use_tc_tiling_on_sc=False, needs_layout_passes=False),
    )(_body)
    return kern(ids3d, table)


def kernel(input_ids, embed_weight):
    ids3d = input_ids.astype(jnp.int32).T.reshape(L, BH, 128)
    lhs5d, pool4d = _encode(ids3d, embed_weight)
    lhs = lhs5d.transpose(2, 4, 0, 1, 3).reshape(B, L, HIDDEN)
    pool = pool4d.transpose(1, 3, 0, 2).reshape(B, HIDDEN)
    return (lhs, pool)
